# Initial kernel scaffold; baseline (speedup 1.0000x reference)
import jax
import jax.numpy as jnp
from jax.experimental import pallas as pl


def kernel(x, centroids, Wv):
    raise NotImplementedError("write your pallas kernel here")


# speedup vs baseline: 1.8211x; 1.8211x over previous
import jax
import jax.numpy as jnp
import numpy as np
from jax.experimental import pallas as pl
from jax.experimental.pallas import tpu as pltpu

B, C, H, W = 4, 96, 224, 224
HW = H * W
S = 196
SPAD = 256
CP = 128
NSAMPLES = 4
T = 1024
NT = HW // T

_ROT0 = (13, 15, 26, 6)
_ROT1 = (17, 29, 16, 24)
_TFC = 0x1BD11BDA


def _np_threefry2x32(k0, k1, x0, x1):
    k0, k1, x0, x1 = (np.uint32(v) for v in (k0, k1, x0, x1))
    ks = [k0, k1, np.uint32(k0 ^ k1 ^ np.uint32(_TFC))]
    x0 = np.uint32((int(x0) + int(k0)) & 0xFFFFFFFF)
    x1 = np.uint32((int(x1) + int(k1)) & 0xFFFFFFFF)
    for g in range(5):
        rots = _ROT0 if g % 2 == 0 else _ROT1
        for r in rots:
            x0 = np.uint32((int(x0) + int(x1)) & 0xFFFFFFFF)
            x1 = np.uint32(((int(x1) << r) | (int(x1) >> (32 - r))) & 0xFFFFFFFF)
            x1 = np.uint32(x1 ^ x0)
        x0 = np.uint32((int(x0) + int(ks[(g + 1) % 3])) & 0xFFFFFFFF)
        x1 = np.uint32((int(x1) + int(ks[(g + 2) % 3]) + g + 1) & 0xFFFFFFFF)
    return x0, x1


_KEYS = [_np_threefry2x32(0, 42, 0, i) for i in range(NSAMPLES)]


def _tf2x32(k0, k1, x1):
    ks0 = np.uint32(k0)
    ks1 = np.uint32(k1)
    ks2 = np.uint32(int(k0) ^ int(k1) ^ _TFC)
    ks = (ks0, ks1, ks2)
    x0 = jnp.full(x1.shape, ks0, jnp.uint32)
    x1 = x1 + ks1
    for g in range(5):
        rots = _ROT0 if g % 2 == 0 else _ROT1
        for r in rots:
            x0 = x0 + x1
            x1 = (x1 << np.uint32(r)) | (x1 >> np.uint32(32 - r))
            x1 = x1 ^ x0
        x0 = x0 + ks[(g + 1) % 3]
        x1 = x1 + np.uint32((int(ks[(g + 2) % 3]) + g + 1) & 0xFFFFFFFF)
    return x0 ^ x1


def _gumbel_from_bits(bits):
    f = jax.lax.bitcast_convert_type((bits >> np.uint32(9)) | np.uint32(0x3F800000),
                                     jnp.float32)
    u = f - 1.0
    tiny = np.float32(np.finfo(np.float32).tiny)
    u = jnp.maximum(tiny, u * (np.float32(1.0) - tiny) + tiny)
    return -jnp.log(-jnp.log(u))


def _sample_scatter_kernel(x_ref, cent_ref, wv_ref, labels_ref, means_ref, acc_ref):
    b = pl.program_id(0)
    t = pl.program_id(1)

    pixt = jnp.transpose(x_ref[0], (1, 0))
    logits = jnp.dot(pixt, cent_ref[...],
                     preferred_element_type=jnp.float32) / jnp.sqrt(jnp.float32(C))
    lane = jax.lax.broadcasted_iota(jnp.int32, (T, SPAD), 1)
    mask = lane < S
    logits = jnp.where(mask, logits, np.float32(-1e30))
    m = jnp.max(logits, axis=1, keepdims=True)
    e = jnp.exp(logits - m)
    p = e / jnp.sum(e, axis=1, keepdims=True)
    logp = jnp.log(p + np.float32(1e-9))

    vpad = jnp.dot(pixt, wv_ref[...], preferred_element_type=jnp.float32)
    cl = jax.lax.broadcasted_iota(jnp.int32, (T, CP), 1)
    vpad = jnp.where(cl == C, np.float32(1.0), vpad)

    row = jax.lax.broadcasted_iota(jnp.int32, (T, SPAD), 0)
    pixbase = b * HW + t * T
    ctr = ((pixbase + row) * S + lane).astype(jnp.uint32)

    partials = []
    for i in range(NSAMPLES):
        g = _gumbel_from_bits(_tf2x32(_KEYS[i][0], _KEYS[i][1], ctr))
        pert = jnp.where(mask, logp + g, np.float32(-1e30))
        lab = jnp.argmax(pert, axis=1).astype(jnp.int32)
        labels_ref[0, i, :] = lab
        onehot = (lane == lab[:, None]).astype(jnp.float32)
        partials.append(jax.lax.dot_general(
            onehot, vpad, (((0,), (0,)), ((), ())),
            preferred_element_type=jnp.float32))
    part = jnp.stack(partials)

    @pl.when(t == 0)
    def _():
        acc_ref[...] = part

    @pl.when(t > 0)
    def _():
        acc_ref[...] += part

    @pl.when(t == NT - 1)
    def _():
        acc = acc_ref[...]
        counts = acc[:, :, C:C + 1]
        means_ref[0] = acc / jnp.maximum(counts, np.float32(1.0))


def _gather_kernel(labels_ref, means_ref, out_ref):
    sub = jax.lax.broadcasted_iota(jnp.int32, (SPAD, T), 0)
    acc = jnp.zeros((CP, T), jnp.float32)
    for i in range(NSAMPLES):
        lab = labels_ref[0, i, :]
        onehot_t = (sub == lab[None, :]).astype(jnp.float32)
        acc += jax.lax.dot_general(
            means_ref[0, i], onehot_t, (((0,), (0,)), ((), ())),
            preferred_element_type=jnp.float32)
    out_ref[0] = acc[:C, :] * np.float32(1.0 / NSAMPLES)


@jax.jit
def kernel(x, centroids, Wv):
    xf = x.reshape(B, C, HW)
    centp = jnp.zeros((C, SPAD), jnp.float32).at[:, :S].set(centroids.T)
    wvp = jnp.zeros((C, CP), jnp.float32).at[:, :C].set(Wv)

    labels, means = pl.pallas_call(
        _sample_scatter_kernel,
        grid=(B, NT),
        in_specs=[
            pl.BlockSpec((1, C, T), lambda b, t: (b, 0, t)),
            pl.BlockSpec((C, SPAD), lambda b, t: (0, 0)),
            pl.BlockSpec((C, CP), lambda b, t: (0, 0)),
        ],
        out_specs=[
            pl.BlockSpec((1, NSAMPLES, T), lambda b, t: (b * NT + t, 0, 0)),
            pl.BlockSpec((1, NSAMPLES, SPAD, CP), lambda b, t: (b, 0, 0, 0)),
        ],
        out_shape=[
            jax.ShapeDtypeStruct((B * NT, NSAMPLES, T), jnp.int32),
            jax.ShapeDtypeStruct((B, NSAMPLES, SPAD, CP), jnp.float32),
        ],
        scratch_shapes=[pltpu.VMEM((NSAMPLES, SPAD, CP), jnp.float32)],
    )(xf, centp, wvp)

    out = pl.pallas_call(
        _gather_kernel,
        grid=(B, NT),
        in_specs=[
            pl.BlockSpec((1, NSAMPLES, T), lambda b, t: (b * NT + t, 0, 0)),
            pl.BlockSpec((1, NSAMPLES, SPAD, CP), lambda b, t: (b, 0, 0, 0)),
        ],
        out_specs=pl.BlockSpec((1, C, T), lambda b, t: (b, 0, t)),
        out_shape=jax.ShapeDtypeStruct((B, C, HW), jnp.float32),
    )(labels, means)

    return out.reshape(B, C, H, W)


# speedup vs baseline: 1.8610x; 1.0219x over previous
import jax
import jax.numpy as jnp
import numpy as np
from jax.experimental import pallas as pl
from jax.experimental.pallas import tpu as pltpu

B, C, H, W = 4, 96, 224, 224
HW = H * W
S = 196
SPAD = 256
CP = 128
NSAMPLES = 4
T = 1024
NT = HW // T

_ROT0 = (13, 15, 26, 6)
_ROT1 = (17, 29, 16, 24)
_TFC = 0x1BD11BDA


def _np_threefry2x32(k0, k1, x0, x1):
    ks = [np.uint32(k0), np.uint32(k1), np.uint32(k0 ^ k1 ^ np.uint32(_TFC))]
    x0 = np.uint32((int(x0) + int(ks[0])) & 0xFFFFFFFF)
    x1 = np.uint32((int(x1) + int(ks[1])) & 0xFFFFFFFF)
    for g in range(5):
        rots = _ROT0 if g % 2 == 0 else _ROT1
        for r in rots:
            x0 = np.uint32((int(x0) + int(x1)) & 0xFFFFFFFF)
            x1 = np.uint32(((int(x1) << r) | (int(x1) >> (32 - r))) & 0xFFFFFFFF)
            x1 = np.uint32(x1 ^ x0)
        x0 = np.uint32((int(x0) + int(ks[(g + 1) % 3])) & 0xFFFFFFFF)
        x1 = np.uint32((int(x1) + int(ks[(g + 2) % 3]) + g + 1) & 0xFFFFFFFF)
    return x0, x1


_KEYS = [_np_threefry2x32(0, 42, 0, i) for i in range(NSAMPLES)]


def _tf2x32(k0, k1, x1):
    ks0 = np.uint32(k0)
    ks1 = np.uint32(k1)
    ks2 = np.uint32(int(k0) ^ int(k1) ^ _TFC)
    ks = (ks0, ks1, ks2)
    x0 = jnp.full(x1.shape, ks0, jnp.uint32)
    x1 = x1 + ks1
    for g in range(5):
        rots = _ROT0 if g % 2 == 0 else _ROT1
        for r in rots:
            x0 = x0 + x1
            x1 = (x1 << np.uint32(r)) | (x1 >> np.uint32(32 - r))
            x1 = x1 ^ x0
        x0 = x0 + ks[(g + 1) % 3]
        x1 = x1 + np.uint32((int(ks[(g + 2) % 3]) + g + 1) & 0xFFFFFFFF)
    return x0 ^ x1


def _neglog_uniform(bits):
    f = jax.lax.bitcast_convert_type((bits >> np.uint32(9)) | np.uint32(0x3F800000),
                                     jnp.float32)
    u = f - 1.0
    tiny = np.float32(np.finfo(np.float32).tiny)
    u = jnp.maximum(tiny, u * (np.float32(1.0) - tiny) + tiny)
    return -jnp.log(u)


def _fused_kernel(x_ref, cent_ref, wv_ref, out_ref, labels_ref, acc_ref):
    b = pl.program_id(0)
    p = pl.program_id(1)
    t = pl.program_id(2)

    @pl.when(p == 0)
    def _sample_and_scatter():
        pixt = jnp.transpose(x_ref[0], (1, 0))
        logits = jnp.dot(pixt, cent_ref[...],
                         preferred_element_type=jnp.float32) / jnp.sqrt(jnp.float32(C))
        lane = jax.lax.broadcasted_iota(jnp.int32, (T, SPAD), 1)
        mask = lane < S
        logits = jnp.where(mask, logits, np.float32(-1e30))
        m = jnp.max(logits, axis=1, keepdims=True)
        el = jnp.exp(logits - m)
        z = jnp.sum(el, axis=1, keepdims=True)
        w = jnp.where(mask, el + np.float32(1e-9) * z, np.float32(0.0))

        vpad = jnp.dot(pixt, wv_ref[...], preferred_element_type=jnp.float32)
        cl = jax.lax.broadcasted_iota(jnp.int32, (T, CP), 1)
        vpad = jnp.where(cl == C, np.float32(1.0), vpad)

        row = jax.lax.broadcasted_iota(jnp.int32, (T, SPAD), 0)
        pixbase = b * HW + t * T
        ctr = ((pixbase + row) * S + lane).astype(jnp.uint32)

        partials = []
        for i in range(NSAMPLES):
            e = _neglog_uniform(_tf2x32(_KEYS[i][0], _KEYS[i][1], ctr))
            lab = jnp.argmax(w / e, axis=1).astype(jnp.int32)
            labels_ref[i, pl.ds(t * T, T)] = lab
            onehot = (lane == lab[:, None]).astype(jnp.float32)
            partials.append(jax.lax.dot_general(
                onehot, vpad, (((0,), (0,)), ((), ())),
                preferred_element_type=jnp.float32))
        part = jnp.stack(partials)

        @pl.when(t == 0)
        def _():
            acc_ref[...] = part

        @pl.when(t > 0)
        def _():
            acc_ref[...] += part

        @pl.when(t == NT - 1)
        def _():
            acc = acc_ref[...]
            counts = acc[:, :, C:C + 1]
            acc_ref[...] = acc / jnp.maximum(counts, np.float32(1.0))

    @pl.when(p == 1)
    def _gather():
        sub = jax.lax.broadcasted_iota(jnp.int32, (SPAD, T), 0)
        gathered = jnp.zeros((CP, T), jnp.float32)
        for i in range(NSAMPLES):
            lab = labels_ref[i, pl.ds(t * T, T)]
            onehot_t = (sub == lab[None, :]).astype(jnp.float32)
            gathered += jax.lax.dot_general(
                acc_ref[i], onehot_t, (((0,), (0,)), ((), ())),
                preferred_element_type=jnp.float32)
        out_ref[0] = gathered[:C, :] * np.float32(1.0 / NSAMPLES)


@jax.jit
def kernel(x, centroids, Wv):
    xf = x.reshape(B, C, HW)
    centp = jnp.zeros((C, SPAD), jnp.float32).at[:, :S].set(centroids.T)
    wvp = jnp.zeros((C, CP), jnp.float32).at[:, :C].set(Wv)

    out = pl.pallas_call(
        _fused_kernel,
        grid=(B, 2, NT),
        in_specs=[
            pl.BlockSpec((1, C, T), lambda b, p, t: (b, 0, t * (1 - p))),
            pl.BlockSpec((C, SPAD), lambda b, p, t: (0, 0)),
            pl.BlockSpec((C, CP), lambda b, p, t: (0, 0)),
        ],
        out_specs=pl.BlockSpec((1, C, T), lambda b, p, t: (b, 0, t * p)),
        out_shape=jax.ShapeDtypeStruct((B, C, HW), jnp.float32),
        scratch_shapes=[
            pltpu.VMEM((NSAMPLES, HW), jnp.int32),
            pltpu.VMEM((NSAMPLES, SPAD, CP), jnp.float32),
        ],
    )(xf, centp, wvp)

    return out.reshape(B, C, H, W)


# speedup vs baseline: 2.4878x; 1.3368x over previous
import jax
import jax.numpy as jnp
import numpy as np
from jax.experimental import pallas as pl
from jax.experimental.pallas import tpu as pltpu

B, C, H, W = 4, 96, 224, 224
HW = H * W
S = 196
CP = 128
NSAMPLES = 4
T = 1024
NT = HW // T

_ROT0 = (13, 15, 26, 6)
_ROT1 = (17, 29, 16, 24)
_TFC = 0x1BD11BDA


def _np_threefry2x32(k0, k1, x0, x1):
    ks = [np.uint32(k0), np.uint32(k1), np.uint32(k0 ^ k1 ^ np.uint32(_TFC))]
    x0 = np.uint32((int(x0) + int(ks[0])) & 0xFFFFFFFF)
    x1 = np.uint32((int(x1) + int(ks[1])) & 0xFFFFFFFF)
    for g in range(5):
        rots = _ROT0 if g % 2 == 0 else _ROT1
        for r in rots:
            x0 = np.uint32((int(x0) + int(x1)) & 0xFFFFFFFF)
            x1 = np.uint32(((int(x1) << r) | (int(x1) >> (32 - r))) & 0xFFFFFFFF)
            x1 = np.uint32(x1 ^ x0)
        x0 = np.uint32((int(x0) + int(ks[(g + 1) % 3])) & 0xFFFFFFFF)
        x1 = np.uint32((int(x1) + int(ks[(g + 2) % 3]) + g + 1) & 0xFFFFFFFF)
    return x0, x1


_KEYS = [_np_threefry2x32(0, 42, 0, i) for i in range(NSAMPLES)]


def _tf2x32(k0, k1, x1):
    ks0 = np.uint32(k0)
    ks1 = np.uint32(k1)
    ks2 = np.uint32(int(k0) ^ int(k1) ^ _TFC)
    ks = (ks0, ks1, ks2)
    x0 = jnp.full(x1.shape, ks0, jnp.uint32)
    x1 = x1 + ks1
    for g in range(5):
        rots = _ROT0 if g % 2 == 0 else _ROT1
        for r in rots:
            x0 = x0 + x1
            x1 = (x1 << np.uint32(r)) | (x1 >> np.uint32(32 - r))
            x1 = x1 ^ x0
        x0 = x0 + ks[(g + 1) % 3]
        x1 = x1 + np.uint32((int(ks[(g + 2) % 3]) + g + 1) & 0xFFFFFFFF)
    return x0 ^ x1


def _neglog_uniform(bits):
    f = jax.lax.bitcast_convert_type((bits >> np.uint32(9)) | np.uint32(0x3F800000),
                                     jnp.float32)
    u = f - 1.0
    tiny = np.float32(np.finfo(np.float32).tiny)
    u = jnp.maximum(tiny, u * (np.float32(1.0) - tiny) + tiny)
    return -jnp.log(u)


def _fused_kernel(x_ref, cent_ref, wvt_ref, out_ref, labels_ref, acc_ref):
    b = pl.program_id(0)
    p = pl.program_id(1)
    t = pl.program_id(2)

    @pl.when(p == 0)
    def _sample_and_scatter():
        xblk = x_ref[0]
        logits = jnp.dot(cent_ref[...], xblk,
                         preferred_element_type=jnp.float32) / jnp.sqrt(jnp.float32(C))
        m = jnp.max(logits, axis=0, keepdims=True)
        el = jnp.exp(logits - m)
        z = jnp.sum(el, axis=0, keepdims=True)
        w = el + np.float32(1e-9) * z

        vt = jnp.dot(wvt_ref[...], xblk,
                     preferred_element_type=jnp.float32)
        crow = jax.lax.broadcasted_iota(jnp.int32, (CP, T), 0)
        vt = jnp.where(crow == C, np.float32(1.0), vt)

        sub = jax.lax.broadcasted_iota(jnp.int32, (S, T), 0)
        lane = jax.lax.broadcasted_iota(jnp.int32, (S, T), 1)
        pixbase = b * HW + t * T
        ctr = ((pixbase + lane) * S + sub).astype(jnp.uint32)

        partials = []
        for i in range(NSAMPLES):
            e = _neglog_uniform(_tf2x32(_KEYS[i][0], _KEYS[i][1], ctr))
            lab = jnp.argmax(w / e, axis=0).astype(jnp.int32)
            labels_ref[i, pl.ds(t * T, T)] = lab
            onehot = (sub == lab[None, :]).astype(jnp.float32)
            partials.append(jax.lax.dot_general(
                onehot, vt, (((1,), (1,)), ((), ())),
                preferred_element_type=jnp.float32))
        part = jnp.stack(partials)

        @pl.when(t == 0)
        def _():
            acc_ref[...] = part

        @pl.when(t > 0)
        def _():
            acc_ref[...] += part

        @pl.when(t == NT - 1)
        def _():
            acc = acc_ref[...]
            counts = acc[:, :, C:C + 1]
            acc_ref[...] = acc / jnp.maximum(counts, np.float32(1.0))

    @pl.when(p == 1)
    def _gather():
        sub = jax.lax.broadcasted_iota(jnp.int32, (S, T), 0)
        gathered = jnp.zeros((CP, T), jnp.float32)
        for i in range(NSAMPLES):
            lab = labels_ref[i, pl.ds(t * T, T)]
            onehot = (sub == lab[None, :]).astype(jnp.float32)
            gathered += jax.lax.dot_general(
                acc_ref[i], onehot, (((0,), (0,)), ((), ())),
                preferred_element_type=jnp.float32)
        out_ref[0] = gathered[:C, :] * np.float32(1.0 / NSAMPLES)


@jax.jit
def kernel(x, centroids, Wv):
    xf = x.reshape(B, C, HW)
    wvt = jnp.zeros((CP, C), jnp.float32).at[:C, :].set(Wv.T)

    out = pl.pallas_call(
        _fused_kernel,
        grid=(B, 2, NT),
        in_specs=[
            pl.BlockSpec((1, C, T), lambda b, p, t: (b, 0, t * (1 - p))),
            pl.BlockSpec((S, C), lambda b, p, t: (0, 0)),
            pl.BlockSpec((CP, C), lambda b, p, t: (0, 0)),
        ],
        out_specs=pl.BlockSpec((1, C, T), lambda b, p, t: (b, 0, t * p)),
        out_shape=jax.ShapeDtypeStruct((B, C, HW), jnp.float32),
        scratch_shapes=[
            pltpu.VMEM((NSAMPLES, HW), jnp.int32),
            pltpu.VMEM((NSAMPLES, S, CP), jnp.float32),
        ],
    )(xf, centroids, wvt)

    return out.reshape(B, C, H, W)


# speedup vs baseline: 2.5367x; 1.0197x over previous
import jax
import jax.numpy as jnp
import numpy as np
from jax.experimental import pallas as pl
from jax.experimental.pallas import tpu as pltpu

B, C, H, W = 4, 96, 224, 224
HW = H * W
S = 196
CP = 128
NSAMPLES = 4
T = 1792
NT = HW // T

_ROT0 = (13, 15, 26, 6)
_ROT1 = (17, 29, 16, 24)
_TFC = 0x1BD11BDA


def _np_threefry2x32(k0, k1, x0, x1):
    ks = [np.uint32(k0), np.uint32(k1), np.uint32(k0 ^ k1 ^ np.uint32(_TFC))]
    x0 = np.uint32((int(x0) + int(ks[0])) & 0xFFFFFFFF)
    x1 = np.uint32((int(x1) + int(ks[1])) & 0xFFFFFFFF)
    for g in range(5):
        rots = _ROT0 if g % 2 == 0 else _ROT1
        for r in rots:
            x0 = np.uint32((int(x0) + int(x1)) & 0xFFFFFFFF)
            x1 = np.uint32(((int(x1) << r) | (int(x1) >> (32 - r))) & 0xFFFFFFFF)
            x1 = np.uint32(x1 ^ x0)
        x0 = np.uint32((int(x0) + int(ks[(g + 1) % 3])) & 0xFFFFFFFF)
        x1 = np.uint32((int(x1) + int(ks[(g + 2) % 3]) + g + 1) & 0xFFFFFFFF)
    return x0, x1


_KEYS = [_np_threefry2x32(0, 42, 0, i) for i in range(NSAMPLES)]


def _tf2x32(k0, k1, x1):
    ks0 = np.uint32(k0)
    ks1 = np.uint32(k1)
    ks2 = np.uint32(int(k0) ^ int(k1) ^ _TFC)
    ks = (ks0, ks1, ks2)
    x0 = jnp.full(x1.shape, ks0, jnp.uint32)
    x1 = x1 + ks1
    for g in range(5):
        rots = _ROT0 if g % 2 == 0 else _ROT1
        for r in rots:
            x0 = x0 + x1
            x1 = (x1 << np.uint32(r)) | (x1 >> np.uint32(32 - r))
            x1 = x1 ^ x0
        x0 = x0 + ks[(g + 1) % 3]
        x1 = x1 + np.uint32((int(ks[(g + 2) % 3]) + g + 1) & 0xFFFFFFFF)
    return x0 ^ x1


def _neglog_uniform(bits):
    f = jax.lax.bitcast_convert_type((bits >> np.uint32(9)) | np.uint32(0x3F800000),
                                     jnp.float32)
    u = f - 1.0
    tiny = np.float32(np.finfo(np.float32).tiny)
    u = jnp.maximum(tiny, u * (np.float32(1.0) - tiny) + tiny)
    return -jnp.log(u)


def _fused_kernel(x_ref, cent_ref, wvt_ref, out_ref, labels_ref, acc_ref):
    b = pl.program_id(0)
    p = pl.program_id(1)
    t = pl.program_id(2)

    @pl.when(p == 0)
    def _sample_and_scatter():
        xblk = x_ref[0]
        logits = jnp.dot(cent_ref[...], xblk,
                         preferred_element_type=jnp.float32) / jnp.sqrt(jnp.float32(C))
        m = jnp.max(logits, axis=0, keepdims=True)
        el = jnp.exp(logits - m)
        z = jnp.sum(el, axis=0, keepdims=True)
        w = el + np.float32(1e-9) * z

        vt = jnp.dot(wvt_ref[...], xblk,
                     preferred_element_type=jnp.float32)
        crow = jax.lax.broadcasted_iota(jnp.int32, (CP, T), 0)
        vt = jnp.where(crow == C, np.float32(1.0), vt)

        sub = jax.lax.broadcasted_iota(jnp.int32, (S, T), 0)
        lane = jax.lax.broadcasted_iota(jnp.int32, (S, T), 1)
        pixbase = b * HW + t * T
        ctr = ((pixbase + lane) * S + sub).astype(jnp.uint32)

        partials = []
        for i in range(NSAMPLES):
            e = _neglog_uniform(_tf2x32(_KEYS[i][0], _KEYS[i][1], ctr))
            lab = jnp.argmax(w / e, axis=0).astype(jnp.int32)
            labels_ref[i, pl.ds(t * T, T)] = lab
            onehot = (sub == lab[None, :]).astype(jnp.float32)
            partials.append(jax.lax.dot_general(
                onehot, vt, (((1,), (1,)), ((), ())),
                preferred_element_type=jnp.float32))
        part = jnp.stack(partials)

        @pl.when(t == 0)
        def _():
            acc_ref[...] = part

        @pl.when(t > 0)
        def _():
            acc_ref[...] += part

        @pl.when(t == NT - 1)
        def _():
            acc = acc_ref[...]
            counts = acc[:, :, C:C + 1]
            acc_ref[...] = acc / jnp.maximum(counts, np.float32(1.0))

    @pl.when(p == 1)
    def _gather():
        sub = jax.lax.broadcasted_iota(jnp.int32, (S, T), 0)
        gathered = jnp.zeros((CP, T), jnp.float32)
        for i in range(NSAMPLES):
            lab = labels_ref[i, pl.ds(t * T, T)]
            onehot = (sub == lab[None, :]).astype(jnp.float32)
            gathered += jax.lax.dot_general(
                acc_ref[i], onehot, (((0,), (0,)), ((), ())),
                preferred_element_type=jnp.float32)
        out_ref[0] = gathered[:C, :] * np.float32(1.0 / NSAMPLES)


@jax.jit
def kernel(x, centroids, Wv):
    xf = x.reshape(B, C, HW)
    wvt = jnp.zeros((CP, C), jnp.float32).at[:C, :].set(Wv.T)

    out = pl.pallas_call(
        _fused_kernel,
        grid=(B, 2, NT),
        in_specs=[
            pl.BlockSpec((1, C, T), lambda b, p, t: (b, 0, t * (1 - p))),
            pl.BlockSpec((S, C), lambda b, p, t: (0, 0)),
            pl.BlockSpec((CP, C), lambda b, p, t: (0, 0)),
        ],
        out_specs=pl.BlockSpec((1, C, T), lambda b, p, t: (b, 0, t * p)),
        out_shape=jax.ShapeDtypeStruct((B, C, HW), jnp.float32),
        scratch_shapes=[
            pltpu.VMEM((NSAMPLES, HW), jnp.int32),
            pltpu.VMEM((NSAMPLES, S, CP), jnp.float32),
        ],
    )(xf, centroids, wvt)

    return out.reshape(B, C, H, W)


# speedup vs baseline: 2.5522x; 1.0061x over previous
import jax
import jax.numpy as jnp
import numpy as np
from jax.experimental import pallas as pl
from jax.experimental.pallas import tpu as pltpu

B, C, H, W = 4, 96, 224, 224
HW = H * W
S = 196
CP = 128
NSAMPLES = 4
T = 1792
NT = HW // T

_ROT0 = (13, 15, 26, 6)
_ROT1 = (17, 29, 16, 24)
_TFC = 0x1BD11BDA


def _np_threefry2x32(k0, k1, x0, x1):
    ks = [np.uint32(k0), np.uint32(k1), np.uint32(k0 ^ k1 ^ np.uint32(_TFC))]
    x0 = np.uint32((int(x0) + int(ks[0])) & 0xFFFFFFFF)
    x1 = np.uint32((int(x1) + int(ks[1])) & 0xFFFFFFFF)
    for g in range(5):
        rots = _ROT0 if g % 2 == 0 else _ROT1
        for r in rots:
            x0 = np.uint32((int(x0) + int(x1)) & 0xFFFFFFFF)
            x1 = np.uint32(((int(x1) << r) | (int(x1) >> (32 - r))) & 0xFFFFFFFF)
            x1 = np.uint32(x1 ^ x0)
        x0 = np.uint32((int(x0) + int(ks[(g + 1) % 3])) & 0xFFFFFFFF)
        x1 = np.uint32((int(x1) + int(ks[(g + 2) % 3]) + g + 1) & 0xFFFFFFFF)
    return x0, x1


_KEYS = [_np_threefry2x32(0, 42, 0, i) for i in range(NSAMPLES)]


def _tf2x32(k0, k1, x1):
    ks0 = np.uint32(k0)
    ks1 = np.uint32(k1)
    ks2 = np.uint32(int(k0) ^ int(k1) ^ _TFC)
    ks = (ks0, ks1, ks2)
    x0 = jnp.full(x1.shape, ks0, jnp.uint32)
    x1 = x1 + ks1
    for g in range(5):
        rots = _ROT0 if g % 2 == 0 else _ROT1
        for r in rots:
            x0 = x0 + x1
            x1 = (x1 << np.uint32(r)) | (x1 >> np.uint32(32 - r))
            x1 = x1 ^ x0
        x0 = x0 + ks[(g + 1) % 3]
        x1 = x1 + np.uint32((int(ks[(g + 2) % 3]) + g + 1) & 0xFFFFFFFF)
    return x0 ^ x1


def _neglog_uniform(bits):
    f = jax.lax.bitcast_convert_type((bits >> np.uint32(9)) | np.uint32(0x3F800000),
                                     jnp.float32)
    tiny = np.float32(np.finfo(np.float32).tiny)
    u = jnp.maximum(f - 1.0, tiny)
    return -jnp.log(u)


def _fused_kernel(x_ref, cent_ref, wvt_ref, out_ref, labels_ref, acc_ref):
    b = pl.program_id(0)
    p = pl.program_id(1)
    t = pl.program_id(2)

    @pl.when(p == 0)
    def _sample_and_scatter():
        xblk = x_ref[0]
        logits = jnp.dot(cent_ref[...], xblk,
                         preferred_element_type=jnp.float32) / jnp.sqrt(jnp.float32(C))
        m = jnp.max(logits, axis=0, keepdims=True)
        el = jnp.exp(logits - m)
        z = jnp.sum(el, axis=0, keepdims=True)
        w = el + np.float32(1e-9) * z

        vt = jnp.dot(wvt_ref[...], xblk,
                     preferred_element_type=jnp.float32)
        crow = jax.lax.broadcasted_iota(jnp.int32, (CP, T), 0)
        vt = jnp.where(crow == C, np.float32(1.0), vt)

        sub = jax.lax.broadcasted_iota(jnp.int32, (S, T), 0)
        lane = jax.lax.broadcasted_iota(jnp.int32, (S, T), 1)
        pixbase = b * HW + t * T
        ctr = ((pixbase + lane) * S + sub).astype(jnp.uint32)

        partials = []
        for i in range(NSAMPLES):
            e = _neglog_uniform(_tf2x32(_KEYS[i][0], _KEYS[i][1], ctr))
            lab = jnp.argmax(w / e, axis=0).astype(jnp.int32)
            labels_ref[i, pl.ds(t * T, T)] = lab
            onehot = (sub == lab[None, :]).astype(jnp.float32)
            partials.append(jax.lax.dot_general(
                onehot, vt, (((1,), (1,)), ((), ())),
                preferred_element_type=jnp.float32))
        part = jnp.stack(partials)

        @pl.when(t == 0)
        def _():
            acc_ref[...] = part

        @pl.when(t > 0)
        def _():
            acc_ref[...] += part

        @pl.when(t == NT - 1)
        def _():
            acc = acc_ref[...]
            counts = acc[:, :, C:C + 1]
            acc_ref[...] = acc / jnp.maximum(counts, np.float32(1.0))

    @pl.when(p == 1)
    def _gather():
        sub = jax.lax.broadcasted_iota(jnp.int32, (S, T), 0)
        gathered = jnp.zeros((CP, T), jnp.float32)
        for i in range(NSAMPLES):
            lab = labels_ref[i, pl.ds(t * T, T)]
            onehot = (sub == lab[None, :]).astype(jnp.float32)
            gathered += jax.lax.dot_general(
                acc_ref[i], onehot, (((0,), (0,)), ((), ())),
                preferred_element_type=jnp.float32)
        out_ref[0] = gathered[:C, :] * np.float32(1.0 / NSAMPLES)


@jax.jit
def kernel(x, centroids, Wv):
    xf = x.reshape(B, C, HW)
    wvt = jnp.zeros((CP, C), jnp.float32).at[:C, :].set(Wv.T)

    out = pl.pallas_call(
        _fused_kernel,
        grid=(B, 2, NT),
        in_specs=[
            pl.BlockSpec((1, C, T), lambda b, p, t: (b, 0, t * (1 - p))),
            pl.BlockSpec((S, C), lambda b, p, t: (0, 0)),
            pl.BlockSpec((CP, C), lambda b, p, t: (0, 0)),
        ],
        out_specs=pl.BlockSpec((1, C, T), lambda b, p, t: (b, 0, t * p)),
        out_shape=jax.ShapeDtypeStruct((B, C, HW), jnp.float32),
        compiler_params=pltpu.CompilerParams(
            dimension_semantics=("parallel", "arbitrary", "arbitrary")),
        scratch_shapes=[
            pltpu.VMEM((NSAMPLES, HW), jnp.int32),
            pltpu.VMEM((NSAMPLES, S, CP), jnp.float32),
        ],
    )(xf, centroids, wvt)

    return out.reshape(B, C, H, W)


# speedup vs baseline: 2.5998x; 1.0187x over previous
import jax
import jax.numpy as jnp
import numpy as np
from jax.experimental import pallas as pl
from jax.experimental.pallas import tpu as pltpu

B, C, H, W = 4, 96, 224, 224
HW = H * W
S = 196
CP = 128
NSAMPLES = 4
T = 1792
NT = HW // T

_ROT0 = (13, 15, 26, 6)
_ROT1 = (17, 29, 16, 24)
_TFC = 0x1BD11BDA


def _np_threefry2x32(k0, k1, x0, x1):
    ks = [np.uint32(k0), np.uint32(k1), np.uint32(k0 ^ k1 ^ np.uint32(_TFC))]
    x0 = np.uint32((int(x0) + int(ks[0])) & 0xFFFFFFFF)
    x1 = np.uint32((int(x1) + int(ks[1])) & 0xFFFFFFFF)
    for g in range(5):
        rots = _ROT0 if g % 2 == 0 else _ROT1
        for r in rots:
            x0 = np.uint32((int(x0) + int(x1)) & 0xFFFFFFFF)
            x1 = np.uint32(((int(x1) << r) | (int(x1) >> (32 - r))) & 0xFFFFFFFF)
            x1 = np.uint32(x1 ^ x0)
        x0 = np.uint32((int(x0) + int(ks[(g + 1) % 3])) & 0xFFFFFFFF)
        x1 = np.uint32((int(x1) + int(ks[(g + 2) % 3]) + g + 1) & 0xFFFFFFFF)
    return x0, x1


_KEYS = [_np_threefry2x32(0, 42, 0, i) for i in range(NSAMPLES)]


def _tf2x32(k0, k1, x1):
    ks0 = np.uint32(k0)
    ks1 = np.uint32(k1)
    ks2 = np.uint32(int(k0) ^ int(k1) ^ _TFC)
    ks = (ks0, ks1, ks2)
    x0 = jnp.full(x1.shape, ks0, jnp.uint32)
    x1 = x1 + ks1
    for g in range(5):
        rots = _ROT0 if g % 2 == 0 else _ROT1
        for r in rots:
            x0 = x0 + x1
            x1 = (x1 << np.uint32(r)) | (x1 >> np.uint32(32 - r))
            x1 = x1 ^ x0
        x0 = x0 + ks[(g + 1) % 3]
        x1 = x1 + np.uint32((int(ks[(g + 2) % 3]) + g + 1) & 0xFFFFFFFF)
    return x0 ^ x1


def _neglog_uniform(bits):
    f = jax.lax.bitcast_convert_type((bits >> np.uint32(9)) | np.uint32(0x3F800000),
                                     jnp.float32)
    tiny = np.float32(np.finfo(np.float32).tiny)
    u = jnp.maximum(f - 1.0, tiny)
    return -jnp.log(u)


def _fused_kernel(x_ref, cent_ref, wvt_ref, out_ref, labels_ref, acc_ref):
    b = pl.program_id(0)
    p = pl.program_id(1)
    t = pl.program_id(2)

    @pl.when(p == 0)
    def _sample_and_scatter():
        xblk = x_ref[0]
        logits = jnp.dot(cent_ref[...], xblk,
                         preferred_element_type=jnp.float32) / jnp.sqrt(jnp.float32(C))
        m = jnp.max(logits, axis=0, keepdims=True)
        el = jnp.exp(logits - m)
        z = jnp.sum(el, axis=0, keepdims=True)
        winv = np.float32(1.0) / (el + np.float32(1e-9) * z)
        sidx = jax.lax.broadcasted_iota(jnp.int32, (1, S), 1).astype(jnp.float32)

        vt = jnp.dot(wvt_ref[...], xblk,
                     preferred_element_type=jnp.float32)
        crow = jax.lax.broadcasted_iota(jnp.int32, (CP, T), 0)
        vt = jnp.where(crow == C, np.float32(1.0), vt)

        sub = jax.lax.broadcasted_iota(jnp.int32, (S, T), 0)
        lane = jax.lax.broadcasted_iota(jnp.int32, (S, T), 1)
        pixbase = b * HW + t * T
        ctr = ((pixbase + lane) * S + sub).astype(jnp.uint32)

        partials = []
        for i in range(NSAMPLES):
            e = _neglog_uniform(_tf2x32(_KEYS[i][0], _KEYS[i][1], ctr))
            r = e * winv
            rmin = jnp.min(r, axis=0, keepdims=True)
            onehot = (r == rmin).astype(jnp.float32)
            lab = jnp.dot(sidx, onehot,
                          preferred_element_type=jnp.float32)
            labels_ref[i, pl.ds(t * T, T)] = lab[0].astype(jnp.int32)
            partials.append(jax.lax.dot_general(
                onehot, vt, (((1,), (1,)), ((), ())),
                preferred_element_type=jnp.float32))
        part = jnp.stack(partials)

        @pl.when(t == 0)
        def _():
            acc_ref[...] = part

        @pl.when(t > 0)
        def _():
            acc_ref[...] += part

        @pl.when(t == NT - 1)
        def _():
            acc = acc_ref[...]
            counts = acc[:, :, C:C + 1]
            acc_ref[...] = acc / jnp.maximum(counts, np.float32(1.0))

    @pl.when(p == 1)
    def _gather():
        sub = jax.lax.broadcasted_iota(jnp.int32, (S, T), 0)
        gathered = jnp.zeros((CP, T), jnp.float32)
        for i in range(NSAMPLES):
            lab = labels_ref[i, pl.ds(t * T, T)]
            onehot = (sub == lab[None, :]).astype(jnp.float32)
            gathered += jax.lax.dot_general(
                acc_ref[i], onehot, (((0,), (0,)), ((), ())),
                preferred_element_type=jnp.float32)
        out_ref[0] = gathered[:C, :] * np.float32(1.0 / NSAMPLES)


@jax.jit
def kernel(x, centroids, Wv):
    xf = x.reshape(B, C, HW)
    wvt = jnp.zeros((CP, C), jnp.float32).at[:C, :].set(Wv.T)

    out = pl.pallas_call(
        _fused_kernel,
        grid=(B, 2, NT),
        in_specs=[
            pl.BlockSpec((1, C, T), lambda b, p, t: (b, 0, t * (1 - p))),
            pl.BlockSpec((S, C), lambda b, p, t: (0, 0)),
            pl.BlockSpec((CP, C), lambda b, p, t: (0, 0)),
        ],
        out_specs=pl.BlockSpec((1, C, T), lambda b, p, t: (b, 0, t * p)),
        out_shape=jax.ShapeDtypeStruct((B, C, HW), jnp.float32),
        compiler_params=pltpu.CompilerParams(
            dimension_semantics=("parallel", "arbitrary", "arbitrary")),
        scratch_shapes=[
            pltpu.VMEM((NSAMPLES, HW), jnp.int32),
            pltpu.VMEM((NSAMPLES, S, CP), jnp.float32),
        ],
    )(xf, centroids, wvt)

    return out.reshape(B, C, H, W)
